# baseline (speedup 1.0000x reference)
import functools

import jax
import jax.numpy as jnp
from jax import lax
from jax.experimental import pallas as pl
from jax.experimental.pallas import tpu as pltpu
from jax.experimental.pallas import tpu_sc as plsc

_NC = 2
_NS = 16
_NW = _NC * _NS

_NBB = 16
_CPAD = 32


@functools.partial(jax.jit, static_argnums=(6, 7, 8, 9))
def _sc_tabular(ftab, xcat_pad, x_num, w_num, b_num, off, B, NNUM, NCAT, D):
    NV = NNUM + NCAT
    bpw = B // _NW
    chunks = bpw // _NBB
    rows = _NBB * NV
    jgroups = D // 16

    mesh = plsc.VectorSubcoreMesh(core_axis_name="c", subcore_axis_name="s")

    @functools.partial(
        pl.kernel,
        mesh=mesh,
        compiler_params=pltpu.CompilerParams(use_tc_tiling_on_sc=True),
        out_type=jax.ShapeDtypeStruct((B, NV, D), jnp.float32),
        scratch_types=[
            pltpu.VMEM((_NBB * _CPAD,), jnp.int32),
            pltpu.VMEM((_NBB * _CPAD,), jnp.int32),
            pltpu.VMEM((_NBB, NV, D), jnp.float32),
            pltpu.VMEM((_NBB, 16), jnp.float32),
            pltpu.VMEM((NNUM, D), jnp.float32),
            pltpu.VMEM((NNUM, D), jnp.float32),
            pltpu.SemaphoreType.DMA,
        ],
    )
    def body(ftab_h, xcat_h, xnum_h, w_h, b_h, off_h, out_h,
             idx_v, off_v, buf, xn_v, wv, bv, gsem):
        wid = lax.axis_index("s") * _NC + lax.axis_index("c")
        base_b = wid * bpw
        pltpu.sync_copy(w_h, wv)
        pltpu.sync_copy(b_h, bv)
        pltpu.sync_copy(off_h, off_v)

        def chunk_body(c, carry):
            b0 = base_b + c * _NBB
            pltpu.sync_copy(xcat_h.at[pl.ds(b0 * _CPAD, _NBB * _CPAD)], idx_v)
            pltpu.sync_copy(xnum_h.at[pl.ds(b0, _NBB)], xn_v)
            for k in range(_NBB * _CPAD // 16):
                sl = pl.ds(k * 16, 16)
                idx_v[sl] = idx_v[sl] + off_v[sl]
            copies = []
            for b in range(_NBB):
                copies.append(
                    pltpu.async_copy(
                        ftab_h.at[idx_v.at[pl.ds(b * _CPAD, NCAT)]],
                        buf.at[b, pl.ds(NNUM, NCAT), :],
                        gsem,
                    )
                )
            for i in range(NNUM):
                wr = [wv[i, pl.ds(j * 16, 16)] for j in range(jgroups)]
                br = [bv[i, pl.ds(j * 16, 16)] for j in range(jgroups)]

                def num_body(b, carry2, wr=wr, br=br, i=i):
                    xrow = xn_v[b, :]
                    xs = xrow[i]
                    for j in range(jgroups):
                        buf[b, i, pl.ds(j * 16, 16)] = xs * wr[j] + br[j]
                    return carry2

                lax.fori_loop(0, _NBB, num_body, 0)
            for cp in copies:
                cp.wait()
            pltpu.sync_copy(buf, out_h.at[pl.ds(b0, _NBB), :, :])
            return carry

        lax.fori_loop(0, chunks, chunk_body, 0)

    return body(ftab, xcat_pad, x_num, w_num, b_num, off)


def kernel(x_num, x_cat, W_num, b_num, tables):
    B, NNUM = x_num.shape
    NCAT = x_cat.shape[1]
    VOCAB, D = tables.shape[1], tables.shape[2]
    ftab = tables.reshape(NCAT * VOCAB, D)
    xnum_pad = jnp.pad(x_num, ((0, 0), (0, 16 - NNUM)))
    xcat_pad = jnp.pad(x_cat, ((0, 0), (0, _CPAD - NCAT))).reshape(-1)
    off1 = jnp.pad(jnp.arange(NCAT, dtype=jnp.int32) * VOCAB,
                   (0, _CPAD - NCAT))
    off = jnp.tile(off1, _NBB)
    K = 4
    Bs = B // K
    outs = [
        _sc_tabular(ftab,
                    lax.dynamic_slice_in_dim(xcat_pad, k * Bs * _CPAD,
                                             Bs * _CPAD),
                    lax.dynamic_slice_in_dim(xnum_pad, k * Bs, Bs),
                    W_num, b_num, off, Bs, NNUM, NCAT, D)
        for k in range(K)
    ]
    return jnp.concatenate(outs, axis=0)


# speedup vs baseline: 1.7501x; 1.7501x over previous
import functools

import jax
import jax.numpy as jnp
from jax import lax
from jax.experimental import pallas as pl
from jax.experimental.pallas import tpu as pltpu
from jax.experimental.pallas import tpu_sc as plsc

_NC = 2
_NS = 16
_NW = _NC * _NS

_NBB = 8
_CPAD = 32


@functools.partial(jax.jit, static_argnums=(6, 7, 8, 9))
def _sc_tabular(ftab, xcat_pad, x_num, w_num, b_num, off, B, NNUM, NCAT, D):
    NV = NNUM + NCAT
    bpw = B // _NW
    chunks = bpw // _NBB
    jgroups = D // 16

    mesh = plsc.VectorSubcoreMesh(core_axis_name="c", subcore_axis_name="s")

    @functools.partial(
        pl.kernel,
        mesh=mesh,
        out_type=jax.ShapeDtypeStruct((B, NV, D), jnp.float32),
        scratch_types=[
            pltpu.VMEM((bpw * _CPAD,), jnp.int32),
            pltpu.VMEM((_NBB * _CPAD,), jnp.int32),
            pltpu.VMEM((bpw * 16,), jnp.float32),
            pltpu.VMEM((_NBB, NV, D), jnp.float32),
            pltpu.VMEM((_NBB, NV, D), jnp.float32),
            pltpu.VMEM((NNUM, D), jnp.float32),
            pltpu.VMEM((NNUM, D), jnp.float32),
            pltpu.SemaphoreType.DMA,
            pltpu.SemaphoreType.DMA,
            pltpu.SemaphoreType.DMA,
        ],
    )
    def body(ftab_h, xcat_h, xnum_h, w_h, b_h, off_h, out_h,
             idx_all, off_v, xn_all, buf0, buf1, wv, bv,
             gsem, wsem0, wsem1):
        wid = lax.axis_index("s") * _NC + lax.axis_index("c")
        base_b = wid * bpw
        pltpu.sync_copy(w_h, wv)
        pltpu.sync_copy(b_h, bv)
        pltpu.sync_copy(off_h, off_v)
        pltpu.sync_copy(xcat_h.at[pl.ds(base_b * _CPAD, bpw * _CPAD)],
                        idx_all)
        pltpu.sync_copy(xnum_h.at[pl.ds(base_b * 16, bpw * 16)], xn_all)

        def do_chunk(c, buf, wsem, wait_write):
            b0 = base_b + c * _NBB
            if wait_write:
                pltpu.make_async_copy(
                    buf, out_h.at[pl.ds(b0, _NBB), :, :], wsem).wait()
            ibase = c * (_NBB * _CPAD)
            for k in range(_NBB * _CPAD // 16):
                sl = pl.ds(ibase + k * 16, 16)
                idx_all[sl] = idx_all[sl] + off_v[pl.ds(k * 16, 16)]
            copies = []
            for b in range(_NBB):
                copies.append(
                    pltpu.async_copy(
                        ftab_h.at[idx_all.at[pl.ds(ibase + b * _CPAD, NCAT)]],
                        buf.at[b, pl.ds(NNUM, NCAT), :],
                        gsem,
                    )
                )
            for i in range(NNUM):
                wr = [wv[i, pl.ds(j * 16, 16)] for j in range(jgroups)]
                br = [bv[i, pl.ds(j * 16, 16)] for j in range(jgroups)]

                def num_body(b, carry2, wr=wr, br=br, i=i):
                    xrow = xn_all[pl.ds((c * _NBB + b) * 16, 16)]
                    xs = xrow[i]
                    for j in range(jgroups):
                        buf[b, i, pl.ds(j * 16, 16)] = xs * wr[j] + br[j]
                    return carry2

                lax.fori_loop(0, _NBB, num_body, 0, unroll=2)
            for cp in copies:
                cp.wait()
            pltpu.async_copy(buf, out_h.at[pl.ds(b0, _NBB), :, :], wsem)

        do_chunk(jnp.int32(0), buf0, wsem0, False)
        do_chunk(jnp.int32(1), buf1, wsem1, False)

        def pair_body(g, carry):
            do_chunk(2 * g, buf0, wsem0, True)
            do_chunk(2 * g + 1, buf1, wsem1, True)
            return carry

        lax.fori_loop(1, chunks // 2, pair_body, 0)
        pltpu.make_async_copy(
            buf0, out_h.at[pl.ds(base_b, _NBB), :, :], wsem0).wait()
        pltpu.make_async_copy(
            buf1, out_h.at[pl.ds(base_b, _NBB), :, :], wsem1).wait()

    return body(ftab, xcat_pad, x_num, w_num, b_num, off)


def kernel(x_num, x_cat, W_num, b_num, tables):
    B, NNUM = x_num.shape
    NCAT = x_cat.shape[1]
    VOCAB, D = tables.shape[1], tables.shape[2]
    ftab = tables.reshape(NCAT * VOCAB, D)
    xnum_pad = jnp.pad(x_num, ((0, 0), (0, 16 - NNUM))).reshape(-1)
    xcat_pad = jnp.pad(x_cat, ((0, 0), (0, _CPAD - NCAT))).reshape(-1)
    off1 = jnp.pad(jnp.arange(NCAT, dtype=jnp.int32) * VOCAB,
                   (0, _CPAD - NCAT))
    off = jnp.tile(off1, _NBB)
    return _sc_tabular(ftab, xcat_pad, xnum_pad, W_num, b_num, off,
                       B, NNUM, NCAT, D)
